# baseline (speedup 1.0000x reference)
import functools

import jax
import jax.numpy as jnp
from jax import lax
from jax.experimental import pallas as pl
from jax.experimental.pallas import tpu as pltpu
from jax.experimental.pallas import tpu_sc as plsc

E = 8
SEQ = 2048
D = 768
DFF = 3072
K = 4096

BT = 256
BF = 512
NF = DFF // BF
NBLK = K // BT + E
NP = NBLK * BT


def _gelu_tanh(v):
    return 0.5 * v * (1.0 + jnp.tanh(jnp.sqrt(2.0 / jnp.pi) * (v + 0.044715 * v ** 3)))


def _scores_body(x_ref, gw_ref, bias_ref, out_ref):
    lg = lax.dot_general(gw_ref[...], x_ref[...], (((1,), (1,)), ((), ())),
                         preferred_element_type=jnp.float32)
    out_ref[...] = jax.nn.sigmoid(lg + bias_ref[...])


def _scores(x_flat, gate_weight, expert_bias):
    return pl.pallas_call(
        _scores_body,
        out_shape=jax.ShapeDtypeStruct((E, SEQ), jnp.float32),
    )(x_flat, gate_weight, expert_bias)


def _mlp_body(be_ref, xg_ref, w1_ref, b1_ref, w2_ref, b2_ref, wp_ref, tok_ref,
              out_ref):
    m = pl.program_id(0)
    used = be_ref[NBLK]

    @pl.when(m == 0)
    def _():
        out_ref[...] = jnp.zeros_like(out_ref)

    @pl.when(m < used)
    def _():
        e = be_ref[m]
        cnt = be_ref[NBLK + 1 + e]
        bstart = be_ref[NBLK + 1 + E + e]
        x_b = xg_ref[...]
        h = lax.dot_general(x_b, w1_ref[0], (((1,), (1,)), ((), ())),
                            preferred_element_type=jnp.float32)
        h = _gelu_tanh(h + b1_ref[0])
        part = lax.dot_general(h, w2_ref[0], (((1,), (1,)), ((), ())),
                               preferred_element_type=jnp.float32)
        r_iota = lax.broadcasted_iota(jnp.int32, (BT, 1), 0)
        valid = (m * BT + r_iota - bstart) < cnt
        w = jnp.where(valid, wp_ref[0, 0][:, None], 0.0)
        y = (part + b2_ref[0]) * w
        toks = tok_ref[0, 0]
        t_iota = lax.broadcasted_iota(jnp.int32, (BT, SEQ), 1)
        onehot = (toks[:, None] == t_iota).astype(jnp.float32)
        out_ref[...] += lax.dot_general(
            onehot, y, (((0,), (0,)), ((), ())),
            preferred_element_type=jnp.float32)


def _grouped_mlp(xg, W1, b1, W2, b2, w_pad, tok_pad, meta):
    grid_spec = pltpu.PrefetchScalarGridSpec(
        num_scalar_prefetch=1,
        grid=(NBLK,),
        in_specs=[
            pl.BlockSpec((BT, D), lambda m, be: (m, 0)),
            pl.BlockSpec((1, DFF, D), lambda m, be: (be[m], 0, 0)),
            pl.BlockSpec((1, 1, DFF), lambda m, be: (be[m], 0, 0)),
            pl.BlockSpec((1, D, DFF), lambda m, be: (be[m], 0, 0)),
            pl.BlockSpec((1, 1, D), lambda m, be: (be[m], 0, 0)),
            pl.BlockSpec((1, 1, BT), lambda m, be: (m, 0, 0)),
            pl.BlockSpec((1, 1, BT), lambda m, be: (m, 0, 0)),
        ],
        out_specs=pl.BlockSpec((SEQ, D), lambda m, be: (0, 0)),
    )
    return pl.pallas_call(
        _mlp_body,
        grid_spec=grid_spec,
        out_shape=jax.ShapeDtypeStruct((SEQ, D), jnp.float32),
    )(meta, xg, W1, b1.reshape(E, 1, DFF), W2, b2.reshape(E, 1, D),
      w_pad.reshape(NBLK, 1, BT), tok_pad.reshape(NBLK, 1, BT))


NTILES = 32
GRPT = NP // NTILES
GCH = 64
GNCH = GRPT // GCH


def _gather_body(x_hbm, tok_hbm, flags_hbm, xg_hbm, idx_v, rows_v, fl_v, sem):
    tile = lax.axis_index("c") * 16 + lax.axis_index("s")
    base = tile * GRPT
    pltpu.sync_copy(flags_hbm.at[pl.ds(tile * 16, 16)], fl_v)
    fl = fl_v[...]
    for i in range(GNCH):
        fi = fl[i]

        @pl.when(fi > 0)
        def _():
            pltpu.sync_copy(tok_hbm.at[pl.ds(base + i * GCH, GCH)], idx_v)
            for j8 in range(GCH // 16):
                v = idx_v[pl.ds(j8 * 16, 16)]
                idx_v[pl.ds(j8 * 16, 16)] = jnp.minimum(
                    jnp.maximum(v, 0), SEQ - 1)
            pltpu.async_copy(x_hbm.at[idx_v], rows_v, sem).wait()
            pltpu.sync_copy(rows_v, xg_hbm.at[pl.ds(base + i * GCH, GCH), :])


def _sc_gather(x_flat, tok_pad, gflags):
    mesh = plsc.VectorSubcoreMesh(core_axis_name="c", subcore_axis_name="s")
    run = pl.kernel(
        _gather_body,
        out_type=jax.ShapeDtypeStruct((NP, D), jnp.float32),
        mesh=mesh,
        scratch_types=[
            pltpu.VMEM((GCH,), jnp.int32),
            pltpu.VMEM((GCH, D), jnp.float32),
            pltpu.VMEM((16,), jnp.int32),
            pltpu.SemaphoreType.DMA,
        ],
    )
    return run(x_flat, tok_pad, gflags.reshape(-1))


PPT = (E * SEQ) // NTILES
PCH = 128
PNCH = PPT // PCH


def _pairs_body(pos_hbm, flat_hbm, tok_hbm, w_hbm, idx_v, val_v, wv_v, sem):
    tile = lax.axis_index("c") * 16 + lax.axis_index("s")
    base = tile * PPT
    lane = jnp.arange(16, dtype=jnp.int32)
    for i in range(PNCH):
        pltpu.sync_copy(pos_hbm.at[pl.ds(base + i * PCH, PCH)], idx_v)
        pltpu.sync_copy(flat_hbm.at[pl.ds(base + i * PCH, PCH)], wv_v)
        t0 = (base + i * PCH) % SEQ
        for j8 in range(PCH // 16):
            val_v[pl.ds(j8 * 16, 16)] = t0 + j8 * 16 + lane
        pltpu.sync_copy(val_v, tok_hbm.at[idx_v])
        pltpu.sync_copy(wv_v, w_hbm.at[idx_v])


def _sc_pair_scatter(scat_pos, flat):
    mesh = plsc.VectorSubcoreMesh(core_axis_name="c", subcore_axis_name="s")
    run = pl.kernel(
        _pairs_body,
        out_type=(jax.ShapeDtypeStruct((NP + 8,), jnp.int32),
                  jax.ShapeDtypeStruct((NP + 8,), jnp.float32)),
        mesh=mesh,
        scratch_types=[
            pltpu.VMEM((PCH,), jnp.int32),
            pltpu.VMEM((PCH,), jnp.int32),
            pltpu.VMEM((PCH,), jnp.float32),
            pltpu.SemaphoreType.DMA,
        ],
    )
    return run(scat_pos, flat)


def kernel(x, gate_weight, expert_bias, W1, b1, W2, b2):
    Bsz, seq, Dm = x.shape
    x_flat = x.reshape(-1, Dm)

    scores = _scores(x_flat, gate_weight, expert_bias)

    flat = scores.reshape(-1)
    thr = lax.top_k(flat, K)[0][-1]
    gt = flat > thr
    n_gt = gt.sum().astype(jnp.int32)
    eq = flat == thr
    eqrank = jnp.cumsum(eq.astype(jnp.int32))
    sel_flat = gt | (eq & (eqrank <= (K - n_gt)))
    sel2d = sel_flat.reshape(E, SEQ).astype(jnp.int32)

    counts = sel2d.sum(axis=1)
    rank_t = jnp.cumsum(sel2d, axis=1)
    nblk_e = (counts + BT - 1) // BT
    cnb_in = jnp.cumsum(nblk_e)
    blk_start = BT * (cnb_in - nblk_e)

    posmat = blk_start[:, None] + rank_t - 1

    scat_pos = jnp.where(sel_flat, posmat.reshape(-1), NP)
    tok_pad, w_pad = _sc_pair_scatter(scat_pos, flat)
    tok_pad = tok_pad[:NP]
    w_pad = w_pad[:NP]

    used = cnb_in[-1]
    bids = jnp.arange(NBLK, dtype=jnp.int32)
    blk_exp = jnp.searchsorted(cnb_in, bids, side='right').astype(jnp.int32)
    e_last = jnp.max(jnp.where(bids < used, blk_exp, -1))
    blk_exp = jnp.where(bids < used, blk_exp, e_last)

    row0 = (jnp.arange(NTILES, dtype=jnp.int32) * GRPT)[:, None] \
        + (jnp.arange(16, dtype=jnp.int32) * GCH)[None, :]
    gflags = ((row0 < used * BT)
              & (jnp.arange(16, dtype=jnp.int32)[None, :] < GNCH)).astype(jnp.int32)

    meta = jnp.concatenate([blk_exp, used[None], counts, blk_start])

    xg = _sc_gather(x_flat, tok_pad, gflags)
    out = _grouped_mlp(xg, W1, b1, W2, b2, w_pad, tok_pad, meta)

    token_each_expert = counts.astype(jnp.float32) / float(K)
    ones_like_mean = jnp.ones((E,), jnp.float32)
    return (out.reshape(Bsz, seq, Dm), token_each_expert, ones_like_mean)


# speedup vs baseline: 8.9333x; 8.9333x over previous
import functools

import jax
import jax.numpy as jnp
from jax import lax
from jax.experimental import pallas as pl
from jax.experimental.pallas import tpu as pltpu
from jax.experimental.pallas import tpu_sc as plsc

E = 8
SEQ = 2048
D = 768
DFF = 3072
K = 4096

BT = 256
BF = 512
NF = DFF // BF
NBLK = K // BT + E
NP = NBLK * BT


def _gelu_tanh(v):
    return 0.5 * v * (1.0 + jnp.tanh(jnp.sqrt(2.0 / jnp.pi) * (v + 0.044715 * v ** 3)))


def _scores_body(x_ref, gw_ref, bias_ref, out_ref):
    lg = lax.dot_general(gw_ref[...], x_ref[...], (((1,), (1,)), ((), ())),
                         preferred_element_type=jnp.float32)
    out_ref[...] = jax.nn.sigmoid(lg + bias_ref[...])


def _scores(x_flat, gate_weight, expert_bias):
    return pl.pallas_call(
        _scores_body,
        out_shape=jax.ShapeDtypeStruct((E, SEQ), jnp.float32),
    )(x_flat, gate_weight, expert_bias)


def _mlp_body(be_ref, xg_ref, w1_ref, b1_ref, w2_ref, b2_ref, wp_ref, tok_ref,
              out_ref):
    m = pl.program_id(0)
    used = be_ref[NBLK]

    @pl.when(m == 0)
    def _():
        out_ref[...] = jnp.zeros_like(out_ref)

    @pl.when(m < used)
    def _():
        e = be_ref[m]
        cnt = be_ref[NBLK + 1 + e]
        bstart = be_ref[NBLK + 1 + E + e]
        x_b = xg_ref[...]
        h = lax.dot_general(x_b, w1_ref[0], (((1,), (1,)), ((), ())),
                            preferred_element_type=jnp.float32)
        h = _gelu_tanh(h + b1_ref[0])
        part = lax.dot_general(h, w2_ref[0], (((1,), (1,)), ((), ())),
                               preferred_element_type=jnp.float32)
        r_iota = lax.broadcasted_iota(jnp.int32, (BT, 1), 0)
        valid = (m * BT + r_iota - bstart) < cnt
        w = jnp.where(valid, wp_ref[0, 0][:, None], 0.0)
        y = (part + b2_ref[0]) * w
        toks = tok_ref[0, 0]
        t_iota = lax.broadcasted_iota(jnp.int32, (BT, SEQ), 1)
        onehot = (toks[:, None] == t_iota).astype(jnp.float32)
        out_ref[...] += lax.dot_general(
            onehot, y, (((0,), (0,)), ((), ())),
            preferred_element_type=jnp.float32)


def _grouped_mlp(xg, W1, b1, W2, b2, w_pad, tok_pad, meta):
    grid_spec = pltpu.PrefetchScalarGridSpec(
        num_scalar_prefetch=1,
        grid=(NBLK,),
        in_specs=[
            pl.BlockSpec((BT, D), lambda m, be: (m, 0)),
            pl.BlockSpec((1, DFF, D), lambda m, be: (be[m], 0, 0)),
            pl.BlockSpec((1, 1, DFF), lambda m, be: (be[m], 0, 0)),
            pl.BlockSpec((1, D, DFF), lambda m, be: (be[m], 0, 0)),
            pl.BlockSpec((1, 1, D), lambda m, be: (be[m], 0, 0)),
            pl.BlockSpec((1, 1, BT), lambda m, be: (m, 0, 0)),
            pl.BlockSpec((1, 1, BT), lambda m, be: (m, 0, 0)),
        ],
        out_specs=pl.BlockSpec((SEQ, D), lambda m, be: (0, 0)),
    )
    return pl.pallas_call(
        _mlp_body,
        grid_spec=grid_spec,
        out_shape=jax.ShapeDtypeStruct((SEQ, D), jnp.float32),
    )(meta, xg, W1, b1.reshape(E, 1, DFF), W2, b2.reshape(E, 1, D),
      w_pad.reshape(NBLK, 1, BT), tok_pad.reshape(NBLK, 1, BT))


NTILES = 32
GRPT = NP // NTILES
GCH = 64
GNCH = GRPT // GCH


def _gather_body(x_hbm, tok_hbm, flags_hbm, xg_hbm, idx_v, rows_v, fl_v, sem):
    tile = lax.axis_index("c") * 16 + lax.axis_index("s")
    base = tile * GRPT
    pltpu.sync_copy(flags_hbm.at[pl.ds(tile * 16, 16)], fl_v)
    fl = fl_v[...]
    for i in range(GNCH):
        fi = fl[i]

        @pl.when(fi > 0)
        def _():
            pltpu.sync_copy(tok_hbm.at[pl.ds(base + i * GCH, GCH)], idx_v)
            for j8 in range(GCH // 16):
                v = idx_v[pl.ds(j8 * 16, 16)]
                idx_v[pl.ds(j8 * 16, 16)] = jnp.minimum(
                    jnp.maximum(v, 0), SEQ - 1)
            pltpu.async_copy(x_hbm.at[idx_v], rows_v, sem).wait()
            pltpu.sync_copy(rows_v, xg_hbm.at[pl.ds(base + i * GCH, GCH), :])


def _sc_gather(x_flat, tok_pad, gflags):
    mesh = plsc.VectorSubcoreMesh(core_axis_name="c", subcore_axis_name="s")
    run = pl.kernel(
        _gather_body,
        out_type=jax.ShapeDtypeStruct((NP, D), jnp.float32),
        mesh=mesh,
        scratch_types=[
            pltpu.VMEM((GCH,), jnp.int32),
            pltpu.VMEM((GCH, D), jnp.float32),
            pltpu.VMEM((16,), jnp.int32),
            pltpu.SemaphoreType.DMA,
        ],
    )
    return run(x_flat, tok_pad, gflags.reshape(-1))


PPT = (E * SEQ) // NTILES
PCH = 128
PNCH = PPT // PCH


def _pairs_body(pos_hbm, flat_hbm, tok_hbm, w_hbm, idx_v, val_v, wv_v, sem):
    tile = lax.axis_index("c") * 16 + lax.axis_index("s")
    base = tile * PPT
    lane = jnp.arange(16, dtype=jnp.int32)
    for i in range(PNCH):
        pltpu.sync_copy(pos_hbm.at[pl.ds(base + i * PCH, PCH)], idx_v)
        pltpu.sync_copy(flat_hbm.at[pl.ds(base + i * PCH, PCH)], wv_v)
        t0 = (base + i * PCH) % SEQ
        for j8 in range(PCH // 16):
            val_v[pl.ds(j8 * 16, 16)] = t0 + j8 * 16 + lane
        pltpu.sync_copy(val_v, tok_hbm.at[idx_v])
        pltpu.sync_copy(wv_v, w_hbm.at[idx_v])


def _sc_pair_scatter(scat_pos, flat):
    mesh = plsc.VectorSubcoreMesh(core_axis_name="c", subcore_axis_name="s")
    run = pl.kernel(
        _pairs_body,
        out_type=(jax.ShapeDtypeStruct((NP + 8,), jnp.int32),
                  jax.ShapeDtypeStruct((NP + 8,), jnp.float32)),
        mesh=mesh,
        scratch_types=[
            pltpu.VMEM((PCH,), jnp.int32),
            pltpu.VMEM((PCH,), jnp.int32),
            pltpu.VMEM((PCH,), jnp.float32),
            pltpu.SemaphoreType.DMA,
        ],
    )
    return run(scat_pos, flat)


def kernel(x, gate_weight, expert_bias, W1, b1, W2, b2):
    Bsz, seq, Dm = x.shape
    x_flat = x.reshape(-1, Dm)

    scores = _scores(x_flat, gate_weight, expert_bias)

    flat = scores.reshape(-1)
    vals, idx = lax.top_k(flat, K)
    e_sel = (idx // SEQ).astype(jnp.int32)
    t_sel = (idx % SEQ).astype(jnp.int32)

    onehot = (e_sel[None, :] == jnp.arange(E, dtype=jnp.int32)[:, None])
    onehot = onehot.astype(jnp.int32)
    counts = onehot.sum(axis=1)
    rank = jnp.cumsum(onehot, axis=1)
    nblk_e = (counts + BT - 1) // BT
    cnb_in = jnp.cumsum(nblk_e)
    blk_start = BT * (cnb_in - nblk_e)

    pos = (onehot * (blk_start[:, None] + rank - 1)).sum(axis=0)
    tok_pad = jnp.zeros((NP,), jnp.int32).at[pos].set(t_sel)
    w_pad = jnp.zeros((NP,), jnp.float32).at[pos].set(vals)

    used = cnb_in[-1]
    bids = jnp.arange(NBLK, dtype=jnp.int32)
    blk_exp = jnp.searchsorted(cnb_in, bids, side='right').astype(jnp.int32)
    e_last = jnp.max(jnp.where(bids < used, blk_exp, -1))
    blk_exp = jnp.where(bids < used, blk_exp, e_last)

    row0 = (jnp.arange(NTILES, dtype=jnp.int32) * GRPT)[:, None] \
        + (jnp.arange(16, dtype=jnp.int32) * GCH)[None, :]
    gflags = ((row0 < used * BT)
              & (jnp.arange(16, dtype=jnp.int32)[None, :] < GNCH)).astype(jnp.int32)

    meta = jnp.concatenate([blk_exp, used[None], counts, blk_start])

    xg = _sc_gather(x_flat, tok_pad, gflags)
    out = _grouped_mlp(xg, W1, b1, W2, b2, w_pad, tok_pad, meta)

    token_each_expert = counts.astype(jnp.float32) / float(K)
    ones_like_mean = jnp.ones((E,), jnp.float32)
    return (out.reshape(Bsz, seq, Dm), token_each_expert, ones_like_mean)
